# Initial kernel scaffold; baseline (speedup 1.0000x reference)
import jax
import jax.numpy as jnp
from jax.experimental import pallas as pl


def kernel(phi_indices, phi_values, phi_inverse_indices, phi_inverse_values, feature_indices, feature_values, weight_matrix, diagonal_weight_filter, dropout):
    raise NotImplementedError("write your pallas kernel here")


# speedup vs baseline: 3.2874x; 3.2874x over previous
import functools

import jax
import jax.numpy as jnp
from jax import lax
from jax.experimental import pallas as pl
from jax.experimental.pallas import tpu as pltpu
from jax.experimental.pallas import tpu_sc as plsc

N = 10000
N_PAD = 10240
D = 128
NC = 2
NS = 16
L = 16
CH = 128


def _make_spmm(e_pad, k_rows):
    per_tile = e_pad // (NC * NS)
    n_chunks = per_tile // CH
    assert n_chunks * CH == per_tile
    rows_per_tile = N_PAD // NS
    wb_rows = 128
    n_wb = rows_per_tile // wb_rows

    mesh = plsc.VectorSubcoreMesh(core_axis_name="c", subcore_axis_name="s")

    @functools.partial(
        pl.kernel,
        mesh=mesh,
        out_type=jax.ShapeDtypeStruct((NC, N_PAD, D), jnp.float32),
        scratch_types=[
            pltpu.VMEM((CH,), jnp.int32),
            pltpu.VMEM((CH,), jnp.int32),
            pltpu.VMEM((CH,), jnp.float32),
            pltpu.VMEM((CH, D), jnp.float32),
            pltpu.VMEM((wb_rows, D), jnp.float32),
            pltpu.VMEM_SHARED((N_PAD, D), jnp.float32),
            pltpu.SemaphoreType.DMA,
        ],
    )
    def spmm(rows_hbm, cols_hbm, vals_hbm, table_hbm, out_hbm,
             cols_v, ridx_v, vals_v, rows_v, stage_v, accum, sem):
        cid = lax.axis_index("c")
        sid = lax.axis_index("s")
        wid = cid * NS + sid

        def zero_body(i, _):
            for d in range(D // L):
                stage_v[i, pl.ds(d * L, L)] = jnp.zeros((L,), jnp.float32)
            return 0
        lax.fori_loop(0, wb_rows, zero_body, 0)
        row0 = sid * rows_per_tile
        for g in range(n_wb):
            pltpu.sync_copy(stage_v, accum.at[pl.ds(row0 + g * wb_rows, wb_rows), :])

        plsc.subcore_barrier()

        def chunk_body(g, _):
            base = wid * per_tile + g * CH
            pltpu.sync_copy(cols_hbm.at[pl.ds(base, CH)], cols_v)
            pltpu.sync_copy(rows_hbm.at[pl.ds(base, CH)], ridx_v)
            pltpu.sync_copy(vals_hbm.at[pl.ds(base, CH)], vals_v)
            pltpu.async_copy(table_hbm.at[cols_v], rows_v, sem).wait()

            def scale_body(g16, _):
                v16 = vals_v[pl.ds(g16 * L, L)]
                for j in range(L):
                    i = g16 * L + j
                    vsp = jnp.full((L,), v16[j], jnp.float32)
                    for d in range(D // L):
                        rows_v[i, pl.ds(d * L, L)] = rows_v[i, pl.ds(d * L, L)] * vsp
                return 0
            lax.fori_loop(0, CH // L, scale_body, 0)

            pltpu.sync_copy(rows_v, accum.at[ridx_v], add=True)
            return 0
        lax.fori_loop(0, n_chunks, chunk_body, 0)

        plsc.subcore_barrier()

        for g in range(n_wb):
            r = row0 + g * wb_rows
            pltpu.sync_copy(accum.at[pl.ds(r, wb_rows), :], stage_v)
            pltpu.sync_copy(stage_v, out_hbm.at[cid, pl.ds(r, wb_rows), :])

    return spmm


def _make_combine(scale_rows, relu):
    rows_per_tile = N_PAD // (NC * NS)
    crows = 64
    n_ch = rows_per_tile // crows

    mesh = plsc.VectorSubcoreMesh(core_axis_name="c", subcore_axis_name="s")

    scratch = [
        pltpu.VMEM((crows, D), jnp.float32),
        pltpu.VMEM((crows, D), jnp.float32),
    ]
    if scale_rows:
        scratch.append(pltpu.VMEM((rows_per_tile,), jnp.float32))

    @functools.partial(
        pl.kernel,
        mesh=mesh,
        out_type=jax.ShapeDtypeStruct((N_PAD, D), jnp.float32),
        scratch_types=scratch,
    )
    def combine(*refs):
        if scale_rows:
            p0_hbm, p1_hbm, th_hbm, out_hbm, buf_a, buf_b, th_v = refs
        else:
            p0_hbm, p1_hbm, out_hbm, buf_a, buf_b = refs
        cid = lax.axis_index("c")
        sid = lax.axis_index("s")
        wid = cid * NS + sid
        row_base = wid * rows_per_tile
        if scale_rows:
            pltpu.sync_copy(th_hbm.at[pl.ds(row_base, rows_per_tile)], th_v)
        for g in range(n_ch):
            r0 = row_base + g * crows
            pltpu.sync_copy(p0_hbm.at[pl.ds(r0, crows), :], buf_a)
            pltpu.sync_copy(p1_hbm.at[pl.ds(r0, crows), :], buf_b)

            def add_body(g16, _):
                if scale_rows:
                    th16 = th_v[pl.ds(g * crows + g16 * L, L)]
                for j in range(L):
                    i = g16 * L + j
                    if scale_rows:
                        th = jnp.full((L,), th16[j], jnp.float32)
                    for d in range(D // L):
                        x = buf_a[i, pl.ds(d * L, L)] + buf_b[i, pl.ds(d * L, L)]
                        if scale_rows:
                            x = x * th
                        if relu:
                            x = jnp.maximum(x, 0.0)
                        buf_a[i, pl.ds(d * L, L)] = x
                return 0
            lax.fori_loop(0, crows // L, add_body, 0)
            pltpu.sync_copy(buf_a, out_hbm.at[pl.ds(r0, crows), :])

    return combine


def _pad_edges(rows, cols, vals, e_pad):
    e = rows.shape[0]
    pad = e_pad - e
    rows = jnp.concatenate([rows, jnp.zeros((pad,), jnp.int32)])
    cols = jnp.concatenate([cols, jnp.zeros((pad,), jnp.int32)])
    vals = jnp.concatenate([vals, jnp.zeros((pad,), jnp.float32)])
    return rows, cols, vals


def kernel(phi_indices, phi_values, phi_inverse_indices, phi_inverse_values,
           feature_indices, feature_values, weight_matrix, diagonal_weight_filter,
           dropout):
    f32 = jnp.float32
    i32 = jnp.int32
    w = weight_matrix.astype(f32)
    theta = diagonal_weight_filter.reshape(-1).astype(f32)
    theta_pad = jnp.concatenate([theta, jnp.zeros((N_PAD - N,), f32)])

    e_feat = 32 * CH * -(-feature_values.shape[0] // (32 * CH))
    e_phi = 32 * CH * -(-phi_values.shape[0] // (32 * CH))

    fr, fc, fv = _pad_edges(feature_indices[0].astype(i32),
                            feature_indices[1].astype(i32),
                            feature_values.astype(f32), e_feat)
    pir, pic, piv = _pad_edges(phi_inverse_indices[0].astype(i32),
                               phi_inverse_indices[1].astype(i32),
                               phi_inverse_values.astype(f32), e_phi)
    pr, pc, pv = _pad_edges(phi_indices[0].astype(i32),
                            phi_indices[1].astype(i32),
                            phi_values.astype(f32), e_phi)

    spmm_w = _make_spmm(e_feat, D)
    spmm_n = _make_spmm(e_phi, N_PAD)
    comb = _make_combine(scale_rows=False, relu=False)
    comb_theta = _make_combine(scale_rows=True, relu=False)
    comb_relu = _make_combine(scale_rows=False, relu=True)

    p_a = spmm_w(fr, fc, fv, w)
    filtered = comb(p_a[0], p_a[1])
    p_b = spmm_n(pir, pic, piv, filtered)
    tmp_scaled = comb_theta(p_b[0], p_b[1], theta_pad)
    p_c = spmm_n(pr, pc, pv, tmp_scaled)
    out = comb_relu(p_c[0], p_c[1])
    return out[:N].reshape(N, 1, D)


# speedup vs baseline: 3.3388x; 1.0156x over previous
import functools

import jax
import jax.numpy as jnp
from jax import lax
from jax.experimental import pallas as pl
from jax.experimental.pallas import tpu as pltpu
from jax.experimental.pallas import tpu_sc as plsc

N = 10000
N_PAD = 10240
D = 128
NC = 2
NS = 16
L = 16
CH = 128


def _make_spmm(e_pad, k_rows):
    per_tile = e_pad // (NC * NS)
    n_chunks = per_tile // CH
    assert n_chunks * CH == per_tile and n_chunks % 4 == 0
    rows_per_tile = N_PAD // NS
    wb_rows = 128
    n_wb = rows_per_tile // wb_rows

    mesh = plsc.VectorSubcoreMesh(core_axis_name="c", subcore_axis_name="s")

    @functools.partial(
        pl.kernel,
        mesh=mesh,
        out_type=jax.ShapeDtypeStruct((NC, N_PAD, D), jnp.float32),
        scratch_types=[
            pltpu.VMEM((4, 2, CH), jnp.int32),
            pltpu.VMEM((4, CH), jnp.float32),
            pltpu.VMEM((CH, D), jnp.float32),
            pltpu.VMEM((CH, D), jnp.float32),
            pltpu.VMEM_SHARED((N_PAD, D), jnp.float32),
            pltpu.SemaphoreType.DMA, pltpu.SemaphoreType.DMA,
            pltpu.SemaphoreType.DMA, pltpu.SemaphoreType.DMA,
            pltpu.SemaphoreType.DMA, pltpu.SemaphoreType.DMA,
            pltpu.SemaphoreType.DMA, pltpu.SemaphoreType.DMA,
        ],
    )
    def spmm(eidx_hbm, evals_hbm, table_hbm, out_hbm,
             ibuf, vbuf, rows_v0, rows_v1, accum,
             isem0, isem1, isem2, isem3, gsem0, gsem1, ssem0, ssem1):
        cid = lax.axis_index("c")
        sid = lax.axis_index("s")
        wid = cid * NS + sid
        rows_v = [rows_v0, rows_v1]
        isem = [isem0, isem1, isem2, isem3]
        gsem = [gsem0, gsem1]
        ssem = [ssem0, ssem1]
        c0 = wid * n_chunks

        def zero_body(i, _):
            for d in range(D // L):
                rows_v0[i, pl.ds(d * L, L)] = jnp.zeros((L,), jnp.float32)
            return 0
        lax.fori_loop(0, wb_rows, zero_body, 0)
        row0 = sid * rows_per_tile
        for g in range(n_wb):
            pltpu.sync_copy(rows_v0, accum.at[pl.ds(row0 + g * wb_rows, wb_rows), :])

        plsc.subcore_barrier()

        def issue_idx(g, slot):
            pltpu.async_copy(eidx_hbm.at[c0 + g], ibuf.at[slot], isem[slot])
            pltpu.async_copy(evals_hbm.at[c0 + g], vbuf.at[slot], isem[slot])

        def wait_idx(slot):
            pltpu.make_async_copy(eidx_hbm.at[0], ibuf.at[slot], isem[slot]).wait()
            pltpu.make_async_copy(evals_hbm.at[0], vbuf.at[slot], isem[slot]).wait()

        def issue_gather(g, rb, slot):
            pltpu.async_copy(table_hbm.at[ibuf.at[slot, 0]], rows_v[rb], gsem[rb])

        def wait_hbm64k(buf, sem):
            pltpu.make_async_copy(out_hbm.at[0, pl.ds(0, CH), :], buf, sem).wait()

        def scale_and_scatter(rb, slot):
            def scale_body(g16, _):
                v16 = vbuf[slot, pl.ds(g16 * L, L)]
                for j in range(L):
                    i = g16 * L + j
                    vsp = jnp.full((L,), v16[j], jnp.float32)
                    for d in range(D // L):
                        rows_v[rb][i, pl.ds(d * L, L)] = (
                            rows_v[rb][i, pl.ds(d * L, L)] * vsp)
                return 0
            lax.fori_loop(0, CH // L, scale_body, 0)
            pltpu.async_copy(rows_v[rb], accum.at[ibuf.at[slot, 1]], ssem[rb],
                             add=True)

        issue_idx(0, 0)
        issue_idx(1, 1)

        def pipe_body(it, _):
            for k in range(4):
                g = it * 4 + k
                rb, rbp = k % 2, (k + 1) % 2
                slot, slotp, slotn = k % 4, (k + 3) % 4, (k + 2) % 4

                @pl.when(jnp.logical_and(g >= 2, g < n_chunks + 2))
                def _():
                    wait_hbm64k(rows_v[rb], ssem[rb])

                @pl.when(g < n_chunks)
                def _():
                    wait_idx(slot)
                    issue_gather(g, rb, slot)

                @pl.when(g + 2 < n_chunks)
                def _():
                    issue_idx(g + 2, slotn)

                @pl.when(jnp.logical_and(g >= 1, g < n_chunks + 1))
                def _():
                    wait_hbm64k(rows_v[rbp], gsem[rbp])
                    scale_and_scatter(rbp, slotp)
            return 0
        lax.fori_loop(0, n_chunks // 4 + 1, pipe_body, 0)

        plsc.subcore_barrier()

        for g in range(n_wb):
            r = row0 + g * wb_rows
            pltpu.sync_copy(accum.at[pl.ds(r, wb_rows), :], rows_v0)
            pltpu.sync_copy(rows_v0, out_hbm.at[cid, pl.ds(r, wb_rows), :])

    return spmm


def _make_combine(scale_rows, relu):
    rows_per_tile = N_PAD // (NC * NS)
    crows = 64
    n_ch = rows_per_tile // crows

    mesh = plsc.VectorSubcoreMesh(core_axis_name="c", subcore_axis_name="s")

    scratch = [
        pltpu.VMEM((crows, D), jnp.float32),
        pltpu.VMEM((crows, D), jnp.float32),
    ]
    if scale_rows:
        scratch.append(pltpu.VMEM((rows_per_tile,), jnp.float32))

    @functools.partial(
        pl.kernel,
        mesh=mesh,
        out_type=jax.ShapeDtypeStruct((N_PAD, D), jnp.float32),
        scratch_types=scratch,
    )
    def combine(*refs):
        if scale_rows:
            p0_hbm, p1_hbm, th_hbm, out_hbm, buf_a, buf_b, th_v = refs
        else:
            p0_hbm, p1_hbm, out_hbm, buf_a, buf_b = refs
        cid = lax.axis_index("c")
        sid = lax.axis_index("s")
        wid = cid * NS + sid
        row_base = wid * rows_per_tile
        if scale_rows:
            pltpu.sync_copy(th_hbm.at[pl.ds(row_base, rows_per_tile)], th_v)
        for g in range(n_ch):
            r0 = row_base + g * crows
            pltpu.sync_copy(p0_hbm.at[pl.ds(r0, crows), :], buf_a)
            pltpu.sync_copy(p1_hbm.at[pl.ds(r0, crows), :], buf_b)

            def add_body(g16, _):
                if scale_rows:
                    th16 = th_v[pl.ds(g * crows + g16 * L, L)]
                for j in range(L):
                    i = g16 * L + j
                    if scale_rows:
                        th = jnp.full((L,), th16[j], jnp.float32)
                    for d in range(D // L):
                        x = buf_a[i, pl.ds(d * L, L)] + buf_b[i, pl.ds(d * L, L)]
                        if scale_rows:
                            x = x * th
                        if relu:
                            x = jnp.maximum(x, 0.0)
                        buf_a[i, pl.ds(d * L, L)] = x
                return 0
            lax.fori_loop(0, crows // L, add_body, 0)
            pltpu.sync_copy(buf_a, out_hbm.at[pl.ds(r0, crows), :])

    return combine


def _pack_edges(indices, vals, e_pad):
    e = vals.shape[0]
    pad = e_pad - e
    rows = jnp.concatenate([indices[0].astype(jnp.int32), jnp.zeros((pad,), jnp.int32)])
    cols = jnp.concatenate([indices[1].astype(jnp.int32), jnp.zeros((pad,), jnp.int32)])
    v = jnp.concatenate([vals.astype(jnp.float32), jnp.zeros((pad,), jnp.float32)])
    return (jnp.stack([cols.reshape(-1, CH), rows.reshape(-1, CH)], axis=1),
            v.reshape(-1, CH))


def kernel(phi_indices, phi_values, phi_inverse_indices, phi_inverse_values,
           feature_indices, feature_values, weight_matrix, diagonal_weight_filter,
           dropout):
    f32 = jnp.float32
    i32 = jnp.int32
    w = weight_matrix.astype(f32)
    theta = diagonal_weight_filter.reshape(-1).astype(f32)
    theta_pad = jnp.concatenate([theta, jnp.zeros((N_PAD - N,), f32)])

    grain = 32 * CH * 4
    e_feat = grain * -(-feature_values.shape[0] // grain)
    e_phi = grain * -(-phi_values.shape[0] // grain)

    feat_i, feat_v = _pack_edges(feature_indices, feature_values, e_feat)
    pinv_i, pinv_v = _pack_edges(phi_inverse_indices, phi_inverse_values, e_phi)
    phi_i, phi_v = _pack_edges(phi_indices, phi_values, e_phi)

    spmm_w = _make_spmm(e_feat, D)
    spmm_n = _make_spmm(e_phi, N_PAD)
    comb = _make_combine(scale_rows=False, relu=False)
    comb_theta = _make_combine(scale_rows=True, relu=False)
    comb_relu = _make_combine(scale_rows=False, relu=True)

    p_a = spmm_w(feat_i, feat_v, w)
    filtered = comb(p_a[0], p_a[1])
    p_b = spmm_n(pinv_i, pinv_v, filtered)
    tmp_scaled = comb_theta(p_b[0], p_b[1], theta_pad)
    p_c = spmm_n(phi_i, phi_v, tmp_scaled)
    out = comb_relu(p_c[0], p_c[1])
    return out[:N].reshape(N, 1, D)


# speedup vs baseline: 3.4290x; 1.0270x over previous
import functools

import jax
import jax.numpy as jnp
from jax import lax
from jax.experimental import pallas as pl
from jax.experimental.pallas import tpu as pltpu
from jax.experimental.pallas import tpu_sc as plsc

N = 10000
N_PAD = 10240
D = 128
NC = 2
NS = 16
L = 16
CH = 64


def _make_spmm(e_pad, k_rows):
    per_tile = e_pad // (NC * NS)
    n_chunks = per_tile // CH
    assert n_chunks * CH == per_tile and n_chunks % 8 == 0
    rows_per_tile = N_PAD // NS
    wb_rows = CH
    n_wb = rows_per_tile // wb_rows

    mesh = plsc.VectorSubcoreMesh(core_axis_name="c", subcore_axis_name="s")

    @functools.partial(
        pl.kernel,
        mesh=mesh,
        out_type=jax.ShapeDtypeStruct((NC, N_PAD, D), jnp.float32),
        scratch_types=[
            pltpu.VMEM((8, 2, CH), jnp.int32),
            pltpu.VMEM((8, CH), jnp.float32),
            pltpu.VMEM((CH, D), jnp.float32),
            pltpu.VMEM((CH, D), jnp.float32),
            pltpu.VMEM((CH, D), jnp.float32),
            pltpu.VMEM((CH, D), jnp.float32),
            pltpu.VMEM_SHARED((N_PAD, D), jnp.float32),
        ] + [pltpu.SemaphoreType.DMA] * 16,
    )
    def spmm(eidx_hbm, evals_hbm, table_hbm, out_hbm,
             ibuf, vbuf, rows_v0, rows_v1, rows_v2, rows_v3, accum, *sems):
        cid = lax.axis_index("c")
        sid = lax.axis_index("s")
        wid = cid * NS + sid
        rows_v = [rows_v0, rows_v1, rows_v2, rows_v3]
        isem = list(sems[0:8])
        gsem = list(sems[8:12])
        ssem = list(sems[12:16])
        c0 = wid * n_chunks

        def zero_body(i, _):
            for d in range(D // L):
                rows_v0[i, pl.ds(d * L, L)] = jnp.zeros((L,), jnp.float32)
            return 0
        lax.fori_loop(0, wb_rows, zero_body, 0)
        row0 = sid * rows_per_tile
        for g in range(n_wb):
            pltpu.sync_copy(rows_v0, accum.at[pl.ds(row0 + g * wb_rows, wb_rows), :])

        plsc.subcore_barrier()

        def issue_idx(g, slot):
            pltpu.async_copy(eidx_hbm.at[c0 + g], ibuf.at[slot], isem[slot])
            pltpu.async_copy(evals_hbm.at[c0 + g], vbuf.at[slot], isem[slot])

        def wait_idx(slot):
            pltpu.make_async_copy(eidx_hbm.at[0], ibuf.at[slot], isem[slot]).wait()
            pltpu.make_async_copy(evals_hbm.at[0], vbuf.at[slot], isem[slot]).wait()

        def issue_gather(g, rb, slot):
            pltpu.async_copy(table_hbm.at[ibuf.at[slot, 0]], rows_v[rb], gsem[rb])

        def wait_hbm64k(buf, sem):
            pltpu.make_async_copy(out_hbm.at[0, pl.ds(0, CH), :], buf, sem).wait()

        def scale_and_scatter(rb, slot):
            def scale_body(g16, _):
                v16 = vbuf[slot, pl.ds(g16 * L, L)]
                for j in range(L):
                    i = g16 * L + j
                    vsp = jnp.full((L,), v16[j], jnp.float32)
                    for d in range(D // L):
                        rows_v[rb][i, pl.ds(d * L, L)] = (
                            rows_v[rb][i, pl.ds(d * L, L)] * vsp)
                return 0
            lax.fori_loop(0, CH // L, scale_body, 0)
            pltpu.async_copy(rows_v[rb], accum.at[ibuf.at[slot, 1]], ssem[rb],
                             add=True)

        DEPTH = 3
        issue_idx(0, 0)
        issue_idx(1, 1)

        def pipe_body(it, _):
            for k in range(8):
                g = it * 8 + k
                rb = k % 4
                rbp = (k + 4 - DEPTH) % 4
                slot = k % 8
                slotp = (k + 8 - DEPTH) % 8
                slotn = (k + 2) % 8

                @pl.when(jnp.logical_and(g >= 4, g < n_chunks + 4))
                def _():
                    wait_hbm64k(rows_v[rb], ssem[rb])

                @pl.when(g < n_chunks)
                def _():
                    wait_idx(slot)
                    issue_gather(g, rb, slot)

                @pl.when(jnp.logical_and(g >= DEPTH, g < n_chunks + DEPTH))
                def _():
                    wait_hbm64k(rows_v[rbp], gsem[rbp])
                    scale_and_scatter(rbp, slotp)

                @pl.when(g + 2 < n_chunks)
                def _():
                    issue_idx(g + 2, slotn)
            return 0
        lax.fori_loop(0, n_chunks // 8 + 1, pipe_body, 0)

        plsc.subcore_barrier()

        for g in range(n_wb):
            r = row0 + g * wb_rows
            pltpu.sync_copy(accum.at[pl.ds(r, wb_rows), :], rows_v0)
            pltpu.sync_copy(rows_v0, out_hbm.at[cid, pl.ds(r, wb_rows), :])

    return spmm


def _make_combine(scale_rows, relu):
    rows_per_tile = N_PAD // (NC * NS)
    crows = 64
    n_ch = rows_per_tile // crows

    mesh = plsc.VectorSubcoreMesh(core_axis_name="c", subcore_axis_name="s")

    scratch = [
        pltpu.VMEM((crows, D), jnp.float32),
        pltpu.VMEM((crows, D), jnp.float32),
    ]
    if scale_rows:
        scratch.append(pltpu.VMEM((rows_per_tile,), jnp.float32))

    @functools.partial(
        pl.kernel,
        mesh=mesh,
        out_type=jax.ShapeDtypeStruct((N_PAD, D), jnp.float32),
        scratch_types=scratch,
    )
    def combine(*refs):
        if scale_rows:
            p0_hbm, p1_hbm, th_hbm, out_hbm, buf_a, buf_b, th_v = refs
        else:
            p0_hbm, p1_hbm, out_hbm, buf_a, buf_b = refs
        cid = lax.axis_index("c")
        sid = lax.axis_index("s")
        wid = cid * NS + sid
        row_base = wid * rows_per_tile
        if scale_rows:
            pltpu.sync_copy(th_hbm.at[pl.ds(row_base, rows_per_tile)], th_v)
        for g in range(n_ch):
            r0 = row_base + g * crows
            pltpu.sync_copy(p0_hbm.at[pl.ds(r0, crows), :], buf_a)
            pltpu.sync_copy(p1_hbm.at[pl.ds(r0, crows), :], buf_b)

            def add_body(g16, _):
                if scale_rows:
                    th16 = th_v[pl.ds(g * crows + g16 * L, L)]
                for j in range(L):
                    i = g16 * L + j
                    if scale_rows:
                        th = jnp.full((L,), th16[j], jnp.float32)
                    for d in range(D // L):
                        x = buf_a[i, pl.ds(d * L, L)] + buf_b[i, pl.ds(d * L, L)]
                        if scale_rows:
                            x = x * th
                        if relu:
                            x = jnp.maximum(x, 0.0)
                        buf_a[i, pl.ds(d * L, L)] = x
                return 0
            lax.fori_loop(0, crows // L, add_body, 0)
            pltpu.sync_copy(buf_a, out_hbm.at[pl.ds(r0, crows), :])

    return combine


def _pack_edges(indices, vals, e_pad):
    e = vals.shape[0]
    pad = e_pad - e
    rows = jnp.concatenate([indices[0].astype(jnp.int32), jnp.zeros((pad,), jnp.int32)])
    cols = jnp.concatenate([indices[1].astype(jnp.int32), jnp.zeros((pad,), jnp.int32)])
    v = jnp.concatenate([vals.astype(jnp.float32), jnp.zeros((pad,), jnp.float32)])
    return (jnp.stack([cols.reshape(-1, CH), rows.reshape(-1, CH)], axis=1),
            v.reshape(-1, CH))


def kernel(phi_indices, phi_values, phi_inverse_indices, phi_inverse_values,
           feature_indices, feature_values, weight_matrix, diagonal_weight_filter,
           dropout):
    f32 = jnp.float32
    i32 = jnp.int32
    w = weight_matrix.astype(f32)
    theta = diagonal_weight_filter.reshape(-1).astype(f32)
    theta_pad = jnp.concatenate([theta, jnp.zeros((N_PAD - N,), f32)])

    grain = 32 * CH * 8
    e_feat = grain * -(-feature_values.shape[0] // grain)
    e_phi = grain * -(-phi_values.shape[0] // grain)

    feat_i, feat_v = _pack_edges(feature_indices, feature_values, e_feat)
    pinv_i, pinv_v = _pack_edges(phi_inverse_indices, phi_inverse_values, e_phi)
    phi_i, phi_v = _pack_edges(phi_indices, phi_values, e_phi)

    spmm_w = _make_spmm(e_feat, D)
    spmm_n = _make_spmm(e_phi, N_PAD)
    comb = _make_combine(scale_rows=False, relu=False)
    comb_theta = _make_combine(scale_rows=True, relu=False)
    comb_relu = _make_combine(scale_rows=False, relu=True)

    p_a = spmm_w(feat_i, feat_v, w)
    filtered = comb(p_a[0], p_a[1])
    p_b = spmm_n(pinv_i, pinv_v, filtered)
    tmp_scaled = comb_theta(p_b[0], p_b[1], theta_pad)
    p_c = spmm_n(phi_i, phi_v, tmp_scaled)
    out = comb_relu(p_c[0], p_c[1])
    return out[:N].reshape(N, 1, D)


# speedup vs baseline: 4.1842x; 1.2202x over previous
import functools

import jax
import jax.numpy as jnp
from jax import lax
from jax.experimental import pallas as pl
from jax.experimental.pallas import tpu as pltpu
from jax.experimental.pallas import tpu_sc as plsc

N = 10000
N_PAD = 10240
D = 128
NC = 2
NS = 16
L = 16
CH = 64


def _make_spmm(e_pad, local_table):
    per_tile = e_pad // (NC * NS)
    n_chunks = per_tile // CH
    assert n_chunks * CH == per_tile and n_chunks % 8 == 0
    rows_per_tile = N_PAD // NS
    n_wb = rows_per_tile // CH
    n_rb = 2 if local_table else 4
    DEPTH = 1 if local_table else 3

    mesh = plsc.VectorSubcoreMesh(core_axis_name="c", subcore_axis_name="s")

    scratch = [
        pltpu.VMEM((8, 2, CH), jnp.int32),
        pltpu.VMEM((8, CH), jnp.float32),
    ]
    scratch += [pltpu.VMEM((CH, D), jnp.float32)] * n_rb
    if local_table:
        scratch.append(pltpu.VMEM((D, D), jnp.float32))
    scratch.append(pltpu.VMEM_SHARED((N_PAD, D), jnp.float32))
    scratch += [pltpu.SemaphoreType.DMA] * (8 + 2 * n_rb)

    @functools.partial(
        pl.kernel,
        mesh=mesh,
        out_type=jax.ShapeDtypeStruct((NC, N_PAD, D), jnp.float32),
        scratch_types=scratch,
    )
    def spmm(eidx_hbm, evals_hbm, table_hbm, out_hbm, ibuf, vbuf, *rest):
        rows_v = list(rest[0:n_rb])
        rest = rest[n_rb:]
        if local_table:
            w_v = rest[0]
            rest = rest[1:]
        accum = rest[0]
        sems = rest[1:]
        isem = list(sems[0:8])
        gsem = list(sems[8:8 + n_rb])
        ssem = list(sems[8 + n_rb:8 + 2 * n_rb])
        cid = lax.axis_index("c")
        sid = lax.axis_index("s")
        wid = cid * NS + sid
        c0 = wid * n_chunks

        def zero_body(i, _):
            for d in range(D // L):
                rows_v[0][i, pl.ds(d * L, L)] = jnp.zeros((L,), jnp.float32)
            return 0
        lax.fori_loop(0, CH, zero_body, 0)
        row0 = sid * rows_per_tile
        for g in range(n_wb):
            pltpu.sync_copy(rows_v[0], accum.at[pl.ds(row0 + g * CH, CH), :])
        if local_table:
            pltpu.sync_copy(table_hbm, w_v)

        plsc.subcore_barrier()

        def issue_idx(g, slot):
            pltpu.async_copy(eidx_hbm.at[c0 + g], ibuf.at[slot], isem[slot])
            pltpu.async_copy(evals_hbm.at[c0 + g], vbuf.at[slot], isem[slot])

        def wait_idx(slot):
            pltpu.make_async_copy(eidx_hbm.at[0], ibuf.at[slot], isem[slot]).wait()
            pltpu.make_async_copy(evals_hbm.at[0], vbuf.at[slot], isem[slot]).wait()

        def wait_rows(buf, sem):
            pltpu.make_async_copy(out_hbm.at[0, pl.ds(0, CH), :], buf, sem).wait()

        def scale_scatter(rb, slot):
            def scale_body(g16, _):
                v16 = vbuf[slot, pl.ds(g16 * L, L)]
                for j in range(L):
                    i = g16 * L + j
                    vsp = jnp.full((L,), v16[j], jnp.float32)
                    for d in range(D // L):
                        rows_v[rb][i, pl.ds(d * L, L)] = (
                            rows_v[rb][i, pl.ds(d * L, L)] * vsp)
                return 0
            lax.fori_loop(0, CH // L, scale_body, 0)
            pltpu.async_copy(rows_v[rb], accum.at[ibuf.at[slot, 1]], ssem[rb],
                             add=True)

        def wmul_scatter(rb, slot):
            def scale_body(g16, _):
                v16 = vbuf[slot, pl.ds(g16 * L, L)]
                c16 = ibuf[slot, 0, pl.ds(g16 * L, L)]
                for j in range(L):
                    i = g16 * L + j
                    vsp = jnp.full((L,), v16[j], jnp.float32)
                    col = c16[j]
                    for d in range(D // L):
                        rows_v[rb][i, pl.ds(d * L, L)] = (
                            w_v[col, pl.ds(d * L, L)] * vsp)
                return 0
            lax.fori_loop(0, CH // L, scale_body, 0)
            pltpu.async_copy(rows_v[rb], accum.at[ibuf.at[slot, 1]], ssem[rb],
                             add=True)

        issue_idx(0, 0)
        issue_idx(1, 1)

        def pipe_body(it, _):
            for k in range(8):
                g = it * 8 + k
                rb = k % n_rb
                rbp = (k + n_rb - DEPTH) % n_rb
                slot = k % 8
                slotp = (k + 8 - DEPTH) % 8
                slotn = (k + 2) % 8

                @pl.when(jnp.logical_and(g >= n_rb, g < n_chunks + n_rb))
                def _():
                    wait_rows(rows_v[rb], ssem[rb])

                if not local_table:
                    @pl.when(g < n_chunks)
                    def _():
                        wait_idx(slot)
                        pltpu.async_copy(table_hbm.at[ibuf.at[slot, 0]],
                                         rows_v[rb], gsem[rb])

                    @pl.when(jnp.logical_and(g >= DEPTH, g < n_chunks + DEPTH))
                    def _():
                        wait_rows(rows_v[rbp], gsem[rbp])
                        scale_scatter(rbp, slotp)
                else:
                    @pl.when(g < n_chunks)
                    def _():
                        wait_idx(slot)
                        wmul_scatter(rb, slot)

                @pl.when(g + 2 < n_chunks)
                def _():
                    issue_idx(g + 2, slotn)
            return 0
        lax.fori_loop(0, n_chunks // 8 + 1, pipe_body, 0)

        plsc.subcore_barrier()

        for g in range(n_wb):
            r = row0 + g * CH
            pltpu.sync_copy(accum.at[pl.ds(r, CH), :], rows_v[0])
            pltpu.sync_copy(rows_v[0], out_hbm.at[cid, pl.ds(r, CH), :])

    return spmm


def _pack_edges(indices, vals, e_pad):
    e = vals.shape[0]
    pad = e_pad - e
    rows = jnp.concatenate([indices[0].astype(jnp.int32), jnp.zeros((pad,), jnp.int32)])
    cols = jnp.concatenate([indices[1].astype(jnp.int32), jnp.zeros((pad,), jnp.int32)])
    v = jnp.concatenate([vals.astype(jnp.float32), jnp.zeros((pad,), jnp.float32)])
    return (jnp.stack([cols.reshape(-1, CH), rows.reshape(-1, CH)], axis=1),
            v.reshape(-1, CH))


def kernel(phi_indices, phi_values, phi_inverse_indices, phi_inverse_values,
           feature_indices, feature_values, weight_matrix, diagonal_weight_filter,
           dropout):
    f32 = jnp.float32
    w = weight_matrix.astype(f32)
    theta = diagonal_weight_filter.reshape(-1).astype(f32)
    theta_pad = jnp.concatenate([theta, jnp.zeros((N_PAD - N,), f32)])

    grain = 32 * CH * 8
    e_feat = grain * -(-feature_values.shape[0] // grain)
    e_phi = grain * -(-phi_values.shape[0] // grain)

    feat_i, feat_v = _pack_edges(feature_indices, feature_values, e_feat)
    pinv_i, pinv_v = _pack_edges(phi_inverse_indices, phi_inverse_values, e_phi)
    phi_i, phi_v = _pack_edges(phi_indices, phi_values, e_phi)

    spmm_w = _make_spmm(e_feat, local_table=True)
    spmm_n = _make_spmm(e_phi, local_table=False)

    p_a = spmm_w(feat_i, feat_v, w)
    filtered = p_a[0] + p_a[1]
    p_b = spmm_n(pinv_i, pinv_v, filtered)
    tmp_scaled = theta_pad[:, None] * (p_b[0] + p_b[1])
    p_c = spmm_n(phi_i, phi_v, tmp_scaled)
    out = jax.nn.relu(p_c[0] + p_c[1])
    return out[:N].reshape(N, 1, D)


# speedup vs baseline: 4.3473x; 1.0390x over previous
import functools

import jax
import jax.numpy as jnp
from jax import lax
from jax.experimental import pallas as pl
from jax.experimental.pallas import tpu as pltpu
from jax.experimental.pallas import tpu_sc as plsc

N = 10000
N_PAD = 10240
D = 128
NC = 2
NS = 16
L = 16
CH = 64


def _make_spmm(e_pad, local_table, split=None):
    per_tile = e_pad // (NC * NS)
    n_chunks = per_tile // CH
    assert n_chunks * CH == per_tile and n_chunks % 8 == 0
    ca, cb = split if split else (n_chunks, n_chunks)
    assert ca + cb == 2 * n_chunks and ca % 8 == 0 and cb % 8 == 0
    rows_per_tile = N_PAD // NS
    n_wb = rows_per_tile // CH
    n_rb = 2 if local_table else 4
    DEPTH = 1 if local_table else 3

    mesh = plsc.VectorSubcoreMesh(core_axis_name="c", subcore_axis_name="s")

    scratch = [
        pltpu.VMEM((8, 2, CH), jnp.int32),
        pltpu.VMEM((8, CH), jnp.float32),
    ]
    scratch += [pltpu.VMEM((CH, D), jnp.float32)] * n_rb
    if local_table:
        scratch.append(pltpu.VMEM((D, D), jnp.float32))
    scratch.append(pltpu.VMEM_SHARED((N_PAD, D), jnp.float32))
    scratch += [pltpu.SemaphoreType.DMA] * (8 + 2 * n_rb)

    @functools.partial(
        pl.kernel,
        mesh=mesh,
        out_type=jax.ShapeDtypeStruct((NC, N_PAD, D), jnp.float32),
        scratch_types=scratch,
    )
    def spmm(eidx_hbm, evals_hbm, table_hbm, out_hbm, ibuf, vbuf, *rest):
        rows_v = list(rest[0:n_rb])
        rest = rest[n_rb:]
        if local_table:
            w_v = rest[0]
            rest = rest[1:]
        accum = rest[0]
        sems = rest[1:]
        isem = list(sems[0:8])
        gsem = list(sems[8:8 + n_rb])
        ssem = list(sems[8 + n_rb:8 + 2 * n_rb])
        cid = lax.axis_index("c")
        sid = lax.axis_index("s")
        nc = jnp.where(cid == 0, ca, cb)
        c0 = jnp.where(cid == 0, sid * ca, NS * ca + sid * cb)

        def zero_body(i, _):
            for d in range(D // L):
                rows_v[0][i, pl.ds(d * L, L)] = jnp.zeros((L,), jnp.float32)
            return 0
        lax.fori_loop(0, CH, zero_body, 0)
        row0 = sid * rows_per_tile
        for g in range(n_wb):
            pltpu.sync_copy(rows_v[0], accum.at[pl.ds(row0 + g * CH, CH), :])
        if local_table:
            pltpu.sync_copy(table_hbm, w_v)

        plsc.subcore_barrier()

        def issue_idx(g, slot):
            pltpu.async_copy(eidx_hbm.at[c0 + g], ibuf.at[slot], isem[slot])
            pltpu.async_copy(evals_hbm.at[c0 + g], vbuf.at[slot], isem[slot])

        def wait_idx(slot):
            pltpu.make_async_copy(eidx_hbm.at[0], ibuf.at[slot], isem[slot]).wait()
            pltpu.make_async_copy(evals_hbm.at[0], vbuf.at[slot], isem[slot]).wait()

        def wait_rows(buf, sem):
            pltpu.make_async_copy(out_hbm.at[0, pl.ds(0, CH), :], buf, sem).wait()

        def scale_scatter(rb, slot):
            def scale_body(g16, _):
                v16 = vbuf[slot, pl.ds(g16 * L, L)]
                for j in range(L):
                    i = g16 * L + j
                    vsp = jnp.full((L,), v16[j], jnp.float32)
                    for d in range(D // L):
                        rows_v[rb][i, pl.ds(d * L, L)] = (
                            rows_v[rb][i, pl.ds(d * L, L)] * vsp)
                return 0
            lax.fori_loop(0, CH // L, scale_body, 0)
            pltpu.async_copy(rows_v[rb], accum.at[ibuf.at[slot, 1]], ssem[rb],
                             add=True)

        def wmul_scatter(rb, slot):
            def scale_body(g16, _):
                v16 = vbuf[slot, pl.ds(g16 * L, L)]
                c16 = ibuf[slot, 0, pl.ds(g16 * L, L)]
                for j in range(L):
                    i = g16 * L + j
                    vsp = jnp.full((L,), v16[j], jnp.float32)
                    col = c16[j]
                    for d in range(D // L):
                        rows_v[rb][i, pl.ds(d * L, L)] = (
                            w_v[col, pl.ds(d * L, L)] * vsp)
                return 0
            lax.fori_loop(0, CH // L, scale_body, 0)
            pltpu.async_copy(rows_v[rb], accum.at[ibuf.at[slot, 1]], ssem[rb],
                             add=True)

        issue_idx(0, 0)
        issue_idx(1, 1)

        def pipe_body(it, _):
            for k in range(8):
                g = it * 8 + k
                rb = k % n_rb
                rbp = (k + n_rb - DEPTH) % n_rb
                slot = k % 8
                slotp = (k + 8 - DEPTH) % 8
                slotn = (k + 2) % 8

                @pl.when(jnp.logical_and(g >= n_rb, g < nc + n_rb))
                def _():
                    wait_rows(rows_v[rb], ssem[rb])

                if not local_table:
                    @pl.when(g < nc)
                    def _():
                        wait_idx(slot)
                        pltpu.async_copy(table_hbm.at[ibuf.at[slot, 0]],
                                         rows_v[rb], gsem[rb])

                    @pl.when(jnp.logical_and(g >= DEPTH, g < nc + DEPTH))
                    def _():
                        wait_rows(rows_v[rbp], gsem[rbp])
                        scale_scatter(rbp, slotp)
                else:
                    @pl.when(g < nc)
                    def _():
                        wait_idx(slot)
                        wmul_scatter(rb, slot)

                @pl.when(g + 2 < nc)
                def _():
                    issue_idx(g + 2, slotn)
            return 0
        lax.fori_loop(0, nc // 8 + 1, pipe_body, 0)

        plsc.subcore_barrier()

        for g in range(n_wb):
            r = row0 + g * CH
            pltpu.sync_copy(accum.at[pl.ds(r, CH), :], rows_v[0])
            pltpu.sync_copy(rows_v[0], out_hbm.at[cid, pl.ds(r, CH), :])

    return spmm


def _pack_edges(indices, vals, e_pad):
    e = vals.shape[0]
    pad = e_pad - e
    rows = jnp.concatenate([indices[0].astype(jnp.int32), jnp.zeros((pad,), jnp.int32)])
    cols = jnp.concatenate([indices[1].astype(jnp.int32), jnp.zeros((pad,), jnp.int32)])
    v = jnp.concatenate([vals.astype(jnp.float32), jnp.zeros((pad,), jnp.float32)])
    return (jnp.stack([cols.reshape(-1, CH), rows.reshape(-1, CH)], axis=1),
            v.reshape(-1, CH))


def kernel(phi_indices, phi_values, phi_inverse_indices, phi_inverse_values,
           feature_indices, feature_values, weight_matrix, diagonal_weight_filter,
           dropout):
    f32 = jnp.float32
    w = weight_matrix.astype(f32)
    theta = diagonal_weight_filter.reshape(-1).astype(f32)
    theta_pad = jnp.concatenate([theta, jnp.zeros((N_PAD - N,), f32)])

    grain = 32 * CH * 8
    e_feat = grain * -(-feature_values.shape[0] // grain)
    e_phi = grain * -(-phi_values.shape[0] // grain)

    feat_i, feat_v = _pack_edges(feature_indices, feature_values, e_feat)
    pinv_i, pinv_v = _pack_edges(phi_inverse_indices, phi_inverse_values, e_phi)
    phi_i, phi_v = _pack_edges(phi_indices, phi_values, e_phi)

    nphi = e_phi // (NC * NS) // CH
    spmm_w = _make_spmm(e_feat, local_table=True)
    spmm_n = _make_spmm(e_phi, local_table=False,
                        split=(nphi + nphi // 2, nphi - nphi // 2))

    p_a = spmm_w(feat_i, feat_v, w)
    filtered = p_a[0] + p_a[1]
    p_b = spmm_n(pinv_i, pinv_v, filtered)
    tmp_scaled = theta_pad[:, None] * (p_b[0] + p_b[1])
    p_c = spmm_n(phi_i, phi_v, tmp_scaled)
    out = jax.nn.relu(p_c[0] + p_c[1])
    return out[:N].reshape(N, 1, D)
